# baseline (speedup 1.0000x reference)
import functools

import jax
import jax.numpy as jnp
from jax import lax
from jax.experimental import pallas as pl
from jax.experimental.pallas import tpu as pltpu
from jax.experimental.pallas import tpu_sc as plsc

_NC = 2
_NS = 16
_NW = _NC * _NS
_LANES = 16


def _node_table_kernel(h_ref, w_ref, b_ref, t_ref):
    s = pl.program_id(0)
    d = h_ref.shape[1]
    acc = jnp.dot(h_ref[...], w_ref[...], preferred_element_type=jnp.float32)
    bias = jnp.where(s == 0, 1.0, 0.0).astype(jnp.float32) * b_ref[...]
    acc = acc + bias
    lo = acc[:, : d // 2].astype(jnp.bfloat16)
    hi = acc[:, d // 2 :].astype(jnp.bfloat16)
    lo32 = lax.convert_element_type(
        lax.bitcast_convert_type(lo, jnp.uint16), jnp.uint32)
    hi32 = lax.convert_element_type(
        lax.bitcast_convert_type(hi, jnp.uint16), jnp.uint32)
    word = jnp.bitwise_or(jnp.left_shift(hi32, 16), lo32)
    t_ref[...] = lax.bitcast_convert_type(word, jnp.int32)


def _build_node_table(h, W12, b, block_n):
    n, d = h.shape
    nb = n // block_n
    return pl.pallas_call(
        _node_table_kernel,
        grid=(2, nb),
        in_specs=[
            pl.BlockSpec((block_n, d), lambda s, j: (j, 0)),
            pl.BlockSpec((d, d), lambda s, j: (s, 0)),
            pl.BlockSpec((1, d), lambda s, j: (0, 0)),
        ],
        out_specs=pl.BlockSpec((block_n, d // 2), lambda s, j: (s * nb + j, 0)),
        out_shape=jax.ShapeDtypeStruct((2 * n, d // 2), jnp.int32),
    )(h, W12, b.reshape(1, d))


def _sc_edge_gather(table, src, dstn, chunk, n_chunks, dw, nbuf):
    e_total = src.shape[0]
    per_w = e_total // _NW
    assert n_chunks % nbuf == 0 and chunk * n_chunks == per_w
    mesh = plsc.VectorSubcoreMesh(
        core_axis_name="c", subcore_axis_name="s",
        num_cores=_NC, num_subcores=_NS)

    scratch = [
        pltpu.VMEM((per_w,), jnp.int32),
        pltpu.VMEM((per_w,), jnp.int32),
    ]
    for _ in range(nbuf):
        scratch += [
            pltpu.VMEM((chunk, dw), jnp.int32),
            pltpu.VMEM((chunk, dw), jnp.int32),
            pltpu.VMEM((chunk, 2 * dw), jnp.float32),
            pltpu.SemaphoreType.DMA,
            pltpu.SemaphoreType.DMA,
        ]

    @functools.partial(
        pl.kernel,
        out_type=jax.ShapeDtypeStruct((e_total, 2 * dw), jnp.float32),
        mesh=mesh,
        compiler_params=pltpu.CompilerParams(
            use_tc_tiling_on_sc=False, needs_layout_passes=False),
        scratch_types=scratch,
    )
    def k(table_hbm, src_hbm, dstn_hbm, out_hbm, si, di, *bufs):
        wid = lax.axis_index("s") * _NC + lax.axis_index("c")
        base_w = wid * per_w
        pltpu.sync_copy(src_hbm.at[pl.ds(base_w, per_w)], si)
        pltpu.sync_copy(dstn_hbm.at[pl.ds(base_w, per_w)], di)
        sets = tuple(tuple(bufs[5 * s: 5 * s + 5]) for s in range(nbuf))

        def gstart(kk, s):
            b0, b1, _, gsem, _ = sets[s]
            pltpu.async_copy(table_hbm.at[si.at[pl.ds(kk * chunk, chunk)]],
                             b0, gsem)
            pltpu.async_copy(table_hbm.at[di.at[pl.ds(kk * chunk, chunk)]],
                             b1, gsem)

        def gwait(kk, s):
            b0, b1, _, gsem, _ = sets[s]
            pltpu.make_async_copy(
                table_hbm.at[si.at[pl.ds(kk * chunk, chunk)]], b0, gsem).wait()
            pltpu.make_async_copy(
                table_hbm.at[di.at[pl.ds(kk * chunk, chunk)]], b1, gsem).wait()

        def wwait(kk, s):
            _, _, ob, _, wsem = sets[s]
            pltpu.make_async_copy(
                ob, out_hbm.at[pl.ds(base_w + kk * chunk, chunk)], wsem).wait()

        for s in range(nbuf):
            gstart(s, s)

        def group_body(kg, carry):
            for s in range(nbuf):
                kk = kg * nbuf + s
                b0, b1, ob, gsem, wsem = sets[s]

                @pl.when(kk >= nbuf)
                def _():
                    wwait(kk - nbuf, s)

                gwait(kk, s)

                def add_row(i, c):
                    for j in range(dw // _LANES):
                        sl = pl.ds(j * _LANES, _LANES)
                        w0 = b0[i, sl]
                        w1 = b1[i, sl]
                        lo = (plsc.bitcast(w0 << 16, jnp.float32)
                              + plsc.bitcast(w1 << 16, jnp.float32))
                        msk = jnp.int32(-65536)
                        hi = (plsc.bitcast(w0 & msk, jnp.float32)
                              + plsc.bitcast(w1 & msk, jnp.float32))
                        ob[i, sl] = lo
                        ob[i, pl.ds(dw + j * _LANES, _LANES)] = hi
                    return c

                lax.fori_loop(0, chunk, add_row, 0, unroll=2)

                @pl.when(kk + nbuf < n_chunks)
                def _():
                    gstart(kk + nbuf, s)

                pltpu.async_copy(
                    ob, out_hbm.at[pl.ds(base_w + kk * chunk, chunk)], wsem)
            return carry

        lax.fori_loop(0, n_chunks // nbuf, group_body, 0)
        for s in range(nbuf):
            wwait(n_chunks - nbuf + s, s)

    return k(table, src, dstn)


def _combine_piece_kernel(e_ref, g_ref, w_ref, y_prev_ref, y_ref):
    del y_prev_ref
    y = jnp.dot(e_ref[...], w_ref[...], preferred_element_type=jnp.float32)
    y = y + g_ref[...]
    y_ref[...] = y * jax.nn.sigmoid(y)


def _combine_piece(e, g, W3, y_prev, piece, block_e):
    e_total, d = e.shape
    e_piece = g.shape[0]
    nb = e_piece // block_e
    off = piece * nb
    args = [e, g, W3]
    in_specs = [
        pl.BlockSpec((block_e, d), lambda j: (off + j, 0)),
        pl.BlockSpec((block_e, d), lambda j: (j, 0)),
        pl.BlockSpec((d, d), lambda j: (0, 0)),
    ]
    aliases = {}
    if y_prev is None:
        y_prev = jnp.zeros((8, d), jnp.float32)
        in_specs.append(pl.BlockSpec(memory_space=pl.ANY))
    else:
        in_specs.append(pl.BlockSpec(memory_space=pl.ANY))
        aliases = {3: 0}
    args.append(y_prev)
    return pl.pallas_call(
        _combine_piece_kernel,
        grid=(nb,),
        in_specs=in_specs,
        out_specs=pl.BlockSpec((block_e, d), lambda j: (off + j, 0)),
        out_shape=jax.ShapeDtypeStruct((e_total, d), jnp.float32),
        input_output_aliases=aliases,
    )(*args)


def kernel(h, e, nbr_list, W, b):
    n, d = h.shape
    e_total = e.shape[0]

    W12 = W[: 2 * d]
    W3 = W[2 * d :]

    src = nbr_list[:, 0]
    dstn = nbr_list[:, 1] + n

    table = _build_node_table(h, W12, b, block_n=2000)

    pieces = 5
    e_piece = e_total // pieces
    chunk = 80
    n_chunks = e_piece // _NW // chunk
    gs = [
        _sc_edge_gather(table,
                        lax.slice_in_dim(src, p * e_piece, (p + 1) * e_piece),
                        lax.slice_in_dim(dstn, p * e_piece, (p + 1) * e_piece),
                        chunk=chunk, n_chunks=n_chunks, dw=d // 2, nbuf=5)
        for p in range(pieces)
    ]
    y = None
    for p in range(pieces):
        y = _combine_piece(e, gs[p], W3, y, piece=p, block_e=2000)
    return y


# speedup vs baseline: 1.1285x; 1.1285x over previous
import functools

import jax
import jax.numpy as jnp
from jax import lax
from jax.experimental import pallas as pl
from jax.experimental.pallas import tpu as pltpu
from jax.experimental.pallas import tpu_sc as plsc

_NC = 2
_NS = 16
_NW = _NC * _NS
_LANES = 16


def _node_table_kernel(h_ref, w_ref, b_ref, t_ref):
    s = pl.program_id(0)
    d = h_ref.shape[1]
    acc = jnp.dot(h_ref[...], w_ref[...], preferred_element_type=jnp.float32)
    bias = jnp.where(s == 0, 1.0, 0.0).astype(jnp.float32) * b_ref[...]
    acc = acc + bias
    lo = acc[:, : d // 2].astype(jnp.bfloat16)
    hi = acc[:, d // 2 :].astype(jnp.bfloat16)
    lo32 = lax.convert_element_type(
        lax.bitcast_convert_type(lo, jnp.uint16), jnp.uint32)
    hi32 = lax.convert_element_type(
        lax.bitcast_convert_type(hi, jnp.uint16), jnp.uint32)
    word = jnp.bitwise_or(jnp.left_shift(hi32, 16), lo32)
    t_ref[...] = lax.bitcast_convert_type(word, jnp.int32)


def _build_node_table(h, W12, b, block_n):
    n, d = h.shape
    nb = n // block_n
    return pl.pallas_call(
        _node_table_kernel,
        grid=(2, nb),
        in_specs=[
            pl.BlockSpec((block_n, d), lambda s, j: (j, 0)),
            pl.BlockSpec((d, d), lambda s, j: (s, 0)),
            pl.BlockSpec((1, d), lambda s, j: (0, 0)),
        ],
        out_specs=pl.BlockSpec((block_n, d // 2), lambda s, j: (s * nb + j, 0)),
        out_shape=jax.ShapeDtypeStruct((2 * n, d // 2), jnp.int32),
    )(h, W12, b.reshape(1, d))


def _sc_edge_gather(table, src, dstn, poff, e_piece, chunk, n_chunks, dw, nbuf):
    per_w = e_piece // _NW
    assert n_chunks % nbuf == 0 and chunk * n_chunks == per_w
    mesh = plsc.VectorSubcoreMesh(
        core_axis_name="c", subcore_axis_name="s",
        num_cores=_NC, num_subcores=_NS)

    scratch = [
        pltpu.VMEM((per_w,), jnp.int32),
        pltpu.VMEM((per_w,), jnp.int32),
    ]
    for _ in range(nbuf):
        scratch += [
            pltpu.VMEM((chunk, dw), jnp.int32),
            pltpu.VMEM((chunk, dw), jnp.int32),
            pltpu.VMEM((chunk, 2 * dw), jnp.float32),
            pltpu.SemaphoreType.DMA,
            pltpu.SemaphoreType.DMA,
        ]

    @functools.partial(
        pl.kernel,
        out_type=jax.ShapeDtypeStruct((e_piece, 2 * dw), jnp.float32),
        mesh=mesh,
        compiler_params=pltpu.CompilerParams(
            use_tc_tiling_on_sc=False, needs_layout_passes=False),
        scratch_types=scratch,
    )
    def k(table_hbm, src_hbm, dstn_hbm, out_hbm, si, di, *bufs):
        wid = lax.axis_index("s") * _NC + lax.axis_index("c")
        base_w = wid * per_w
        pltpu.sync_copy(src_hbm.at[pl.ds(poff + base_w, per_w)], si)
        pltpu.sync_copy(dstn_hbm.at[pl.ds(poff + base_w, per_w)], di)
        sets = tuple(tuple(bufs[5 * s: 5 * s + 5]) for s in range(nbuf))

        def gstart(kk, s):
            b0, b1, _, gsem, _ = sets[s]
            pltpu.async_copy(table_hbm.at[si.at[pl.ds(kk * chunk, chunk)]],
                             b0, gsem)
            pltpu.async_copy(table_hbm.at[di.at[pl.ds(kk * chunk, chunk)]],
                             b1, gsem)

        def gwait(kk, s):
            b0, b1, _, gsem, _ = sets[s]
            pltpu.make_async_copy(
                table_hbm.at[si.at[pl.ds(kk * chunk, chunk)]], b0, gsem).wait()
            pltpu.make_async_copy(
                table_hbm.at[di.at[pl.ds(kk * chunk, chunk)]], b1, gsem).wait()

        def wwait(kk, s):
            _, _, ob, _, wsem = sets[s]
            pltpu.make_async_copy(
                ob, out_hbm.at[pl.ds(base_w + kk * chunk, chunk)], wsem).wait()

        for s in range(nbuf):
            gstart(s, s)

        def group_body(kg, carry):
            for s in range(nbuf):
                kk = kg * nbuf + s
                b0, b1, ob, gsem, wsem = sets[s]

                @pl.when(kk >= nbuf)
                def _():
                    wwait(kk - nbuf, s)

                gwait(kk, s)

                @plsc.parallel_loop(0, chunk, unroll=8)
                def _(i):
                    for j in range(dw // _LANES):
                        sl = pl.ds(j * _LANES, _LANES)
                        w0 = b0[i, sl]
                        w1 = b1[i, sl]
                        lo = (plsc.bitcast(w0 << 16, jnp.float32)
                              + plsc.bitcast(w1 << 16, jnp.float32))
                        msk = jnp.int32(-65536)
                        hi = (plsc.bitcast(w0 & msk, jnp.float32)
                              + plsc.bitcast(w1 & msk, jnp.float32))
                        ob[i, sl] = lo
                        ob[i, pl.ds(dw + j * _LANES, _LANES)] = hi

                @pl.when(kk + nbuf < n_chunks)
                def _():
                    gstart(kk + nbuf, s)

                pltpu.async_copy(
                    ob, out_hbm.at[pl.ds(base_w + kk * chunk, chunk)], wsem)
            return carry

        lax.fori_loop(0, n_chunks // nbuf, group_body, 0)
        for s in range(nbuf):
            wwait(n_chunks - nbuf + s, s)

    return k(table, src, dstn)


def _combine_piece_kernel(e_ref, g_ref, w_ref, y_prev_ref, y_ref):
    del y_prev_ref
    y = jnp.dot(e_ref[...].astype(jnp.bfloat16),
                w_ref[...].astype(jnp.bfloat16),
                preferred_element_type=jnp.float32)
    y = y + g_ref[...]
    y_ref[...] = y * jax.nn.sigmoid(y)


def _combine_piece(e, g, W3, y_prev, piece, block_e):
    e_total, d = e.shape
    e_piece = g.shape[0]
    nb = e_piece // block_e
    off = piece * nb
    args = [e, g, W3]
    in_specs = [
        pl.BlockSpec((block_e, d), lambda j: (off + j, 0)),
        pl.BlockSpec((block_e, d), lambda j: (j, 0)),
        pl.BlockSpec((d, d), lambda j: (0, 0)),
    ]
    aliases = {}
    if y_prev is None:
        y_prev = jnp.zeros((8, d), jnp.float32)
        in_specs.append(pl.BlockSpec(memory_space=pl.ANY))
    else:
        in_specs.append(pl.BlockSpec(memory_space=pl.ANY))
        aliases = {3: 0}
    args.append(y_prev)
    return pl.pallas_call(
        _combine_piece_kernel,
        grid=(nb,),
        in_specs=in_specs,
        out_specs=pl.BlockSpec((block_e, d), lambda j: (off + j, 0)),
        out_shape=jax.ShapeDtypeStruct((e_total, d), jnp.float32),
        input_output_aliases=aliases,
    )(*args)


def kernel(h, e, nbr_list, W, b):
    n, d = h.shape
    e_total = e.shape[0]

    W12 = W[: 2 * d]
    W3 = W[2 * d :]

    src = nbr_list[:, 0]
    dstn = nbr_list[:, 1] + n

    table = _build_node_table(h, W12, b, block_n=2000)

    pieces = 5
    e_piece = e_total // pieces
    chunk = 80
    n_chunks = e_piece // _NW // chunk
    gs = [
        _sc_edge_gather(table, src, dstn, poff=p * e_piece, e_piece=e_piece,
                        chunk=chunk, n_chunks=n_chunks, dw=d // 2, nbuf=5)
        for p in range(pieces)
    ]
    y = None
    for p in range(pieces):
        y = _combine_piece(e, gs[p], W3, y, piece=p, block_e=2000)
    return y
